# baseline (speedup 1.0000x reference)
import functools

import jax
import jax.numpy as jnp
from jax import lax
from jax.experimental import pallas as pl
from jax.experimental.pallas import tpu as pltpu
from jax.experimental.pallas import tpu_sc as plsc

_NC = 2
_NS = 16
_NW = _NC * _NS

_B = 1024
_S = 200
_D = 64
_ROWS = _B * _S
_RPW = _ROWS // _NW
_IDXW = 128
_NSTREAM = _RPW // _IDXW
_CH_STREAMS = 10
_CH_ROWS = _CH_STREAMS * _IDXW
_NCH = _NSTREAM // _CH_STREAMS

_EPS = 1e-5


def _sc_gather(table, idx3d):
    mesh = plsc.VectorSubcoreMesh(core_axis_name="c", subcore_axis_name="s")

    @functools.partial(
        pl.kernel,
        mesh=mesh,
        compiler_params=pltpu.CompilerParams(use_tc_tiling_on_sc=False),
        out_type=jax.ShapeDtypeStruct((_ROWS, _D), jnp.float32),
        scratch_types=[
            pltpu.VMEM((_NSTREAM, _IDXW), jnp.int32),
            pltpu.VMEM((_CH_ROWS, _D), jnp.float32),
            pltpu.SemaphoreType.DMA,
        ],
    )
    def k(table_hbm, idx_hbm, out_hbm, idx_v, rows_v, sem):
        wid = lax.axis_index("s") * _NC + lax.axis_index("c")
        pltpu.sync_copy(idx_hbm.at[wid], idx_v)
        base = wid * _RPW
        for g in range(_NCH):
            handles = []
            for j in range(_CH_STREAMS):
                handles.append(pltpu.async_copy(
                    table_hbm.at[idx_v.at[g * _CH_STREAMS + j]],
                    rows_v.at[pl.ds(j * _IDXW, _IDXW)],
                    sem,
                ))
            for h in handles:
                h.wait()
            pltpu.sync_copy(
                rows_v, out_hbm.at[pl.ds(base + g * _CH_ROWS, _CH_ROWS)])

    return k(table, idx3d)


_BB = 8
_BLK = _BB * _S


def _ln_block(x_ref, pe_ref, g_ref, b_ref, o_ref):
    x = x_ref[...] + pe_ref[...]
    m = jnp.mean(x, axis=-1, keepdims=True)
    c = x - m
    v = jnp.mean(c * c, axis=-1, keepdims=True)
    y = c * lax.rsqrt(v + _EPS)
    o_ref[...] = y * g_ref[...] + b_ref[...]


def _tc_layernorm(gathered, pe_tile, gamma2, beta2):
    grid = _ROWS // _BLK
    return pl.pallas_call(
        _ln_block,
        grid=(grid,),
        in_specs=[
            pl.BlockSpec((_BLK, _D), lambda i: (i, 0)),
            pl.BlockSpec((_BLK, _D), lambda i: (0, 0)),
            pl.BlockSpec((1, _D), lambda i: (0, 0)),
            pl.BlockSpec((1, _D), lambda i: (0, 0)),
        ],
        out_specs=pl.BlockSpec((_BLK, _D), lambda i: (i, 0)),
        out_shape=jax.ShapeDtypeStruct((_ROWS, _D), jnp.float32),
    )(gathered, pe_tile, gamma2, beta2)


def kernel(input_ids, table, gamma, beta, pos_enc):
    ids = input_ids.reshape(-1).astype(jnp.int32)
    idx3d = ids.reshape(_NW, _NSTREAM, _IDXW)
    gathered = _sc_gather(table, idx3d)
    pe = pos_enc[0, :_S, :]
    pe_tile = jnp.tile(pe, (_BB, 1))
    out = _tc_layernorm(gathered, pe_tile,
                        gamma.reshape(1, _D), beta.reshape(1, _D))
    return out.reshape(_B, _S, _D)
